# baseline (speedup 1.0000x reference)
import functools

import jax
import jax.numpy as jnp
from jax import lax
from jax.experimental import pallas as pl
from jax.experimental.pallas import tpu as pltpu
from jax.experimental.pallas import tpu_sc as plsc

D_MODEL = 32
D_VOCAB = 1000000
BATCH = 16384
HIST = 20

NC = 2
NS = 16
D_PER_C = D_MODEL // NC
B_PER_T = BATCH // NS


_VC = 16384
_CHUNKS = _VC // 128
_NV = (D_VOCAB + _VC - 1) // _VC
ROW_CHUNKS = _NV * _CHUNKS
ROW_STRIDE = ROW_CHUNKS * 128


def _retile_body(w_ref, out_ref):
    out_ref[...] = w_ref[...].reshape(8, _CHUNKS, 128)


def _retile(w):
    return pl.pallas_call(
        _retile_body,
        grid=(D_MODEL // 8, _NV),
        in_specs=[pl.BlockSpec((8, _VC), lambda r, i: (r, i))],
        out_specs=pl.BlockSpec((8, _CHUNKS, 128), lambda r, i: (r, i, 0)),
        out_shape=jax.ShapeDtypeStruct((D_MODEL, ROW_CHUNKS, 128), jnp.float32),
    )(w)


@functools.partial(
    pl.kernel,
    mesh=plsc.VectorSubcoreMesh(core_axis_name="c", subcore_axis_name="s"),
    out_type=jax.ShapeDtypeStruct((HIST, D_MODEL, BATCH), jnp.float32),
    scratch_types=[
        pltpu.VMEM_SHARED((ROW_STRIDE,), jnp.float32),
        pltpu.VMEM((HIST, B_PER_T), jnp.int32),
        pltpu.VMEM((HIST, B_PER_T), jnp.float32),
        pltpu.SemaphoreType.DMA,
        pltpu.SemaphoreType.DMA,
    ],
    compiler_params=pltpu.CompilerParams(use_tc_tiling_on_sc=False),
)
def _sc_embed(w_hbm, idx_hbm, out_hbm, row_sp, idx_v, buf_v, gsem, wsem):
    c = lax.axis_index("c")
    s = lax.axis_index("s")
    pltpu.sync_copy(idx_hbm.at[s], idx_v)

    def step(dd, carry):
        d = c * D_PER_C + dd

        @pl.when(s == 0)
        def _load_row():
            pltpu.sync_copy(w_hbm.at[pl.ds(d * ROW_STRIDE, ROW_STRIDE)], row_sp)

        plsc.subcore_barrier()
        gathers = [
            pltpu.async_copy(row_sp.at[idx_v.at[p]], buf_v.at[p], gsem)
            for p in range(HIST)
        ]
        for g in gathers:
            g.wait()
        writes = [
            pltpu.async_copy(
                buf_v.at[p], out_hbm.at[p, d, pl.ds(s * B_PER_T, B_PER_T)], wsem
            )
            for p in range(HIST)
        ]
        for w in writes:
            w.wait()
        plsc.subcore_barrier()
        return carry

    lax.fori_loop(0, D_PER_C, step, 0)


def kernel(x, W_E):
    x3 = x.T.reshape(HIST, NS, B_PER_T).transpose(1, 0, 2)
    w_flat = _retile(W_E).reshape(D_MODEL * ROW_STRIDE)
    out_sc = _sc_embed(w_flat, x3)
    return jnp.transpose(out_sc, (2, 0, 1))
